# baseline (speedup 1.0000x reference)
import jax
import jax.numpy as jnp
from jax import lax
from jax.experimental import pallas as pl
from jax.experimental.pallas import tpu as pltpu
from jax.experimental.pallas import tpu_sc as plsc

_VOCAB = 1000000
_SEQ = 200
_DIM = 32
_BATCH = 4096

_NC = 2
_NS = 16
_NW = _NC * _NS
_BPW = _BATCH // _NW
_OSTRIDE = _BPW + 1
_PSTRIDE = _SEQ + 1


def _gather_body(idx_t, word_hbm, pos_tt, out_t,
                 idx_v, pos_v, g0, g1, o0, o1, gsem0, gsem1, osem0, osem1):
    wid = lax.axis_index("s") * _NC + lax.axis_index("c")
    b0 = wid * _BPW
    grows = (g0, g1)
    oblk = (o0, o1)
    gsem = (gsem0, gsem1)
    osem = (osem0, osem1)

    pltpu.sync_copy(idx_t.at[:, pl.ds(b0, _BPW)], idx_v)
    pltpu.sync_copy(pos_tt, pos_v.at[:, pl.ds(0, _SEQ)])

    def g_copy(s, b):
        return pltpu.make_async_copy(
            word_hbm.at[idx_v.at[s]],
            grows[b],
            gsem[b],
        )

    def o_copy(s, b):
        return pltpu.make_async_copy(
            oblk[b].at[:, pl.ds(0, _BPW)],
            out_t.at[s, :, pl.ds(b0, _BPW)],
            osem[b],
        )

    lanes = lax.iota(jnp.int32, 16)

    def compute(s, b):
        gb, ob = grows[b], oblk[b]
        ss = jnp.full((16,), s, jnp.int32)
        pcol = []
        for h in range(2):
            cc = lanes + (16 * h)
            pcol.append(plsc.load_gather(pos_v, [cc, ss]))
        def rbody(r):
            rr = jnp.full((16,), r, jnp.int32)
            for h in range(2):
                cc = lanes + (16 * h)
                vec = gb[r, pl.ds(16 * h, 16)]
                plsc.store_scatter(ob, [cc, rr], vec + pcol[h])

        plsc.parallel_loop(0, _BPW, 1, unroll=8)(rbody)

    g_copy(0, 0).start()

    def s_iter(t, carry):
        for b in range(2):
            s = t * 2 + b
            nb = 1 - b

            @pl.when(s + 1 < _SEQ)
            def _prefetch():
                @pl.when(s >= 1)
                def _drain():
                    o_copy(s - 1, nb).wait()

                g_copy(s + 1, nb).start()

            g_copy(s, b).wait()
            compute(s, b)
            o_copy(s, b).start()
        return carry

    lax.fori_loop(0, _SEQ // 2, s_iter, 0)

    o_copy(_SEQ - 2, 0).wait()
    o_copy(_SEQ - 1, 1).wait()


@jax.jit
def _embed(inputs, word_table, pos_table):
    mesh = plsc.VectorSubcoreMesh(core_axis_name="c", subcore_axis_name="s")
    idx_t = inputs.T
    pos_tt = pos_table.T

    word128 = lax.optimization_barrier(word_table.reshape(250000, 128))
    word_lin = word128.reshape(_VOCAB, _DIM)

    out_t = pl.kernel(
        _gather_body,
        out_type=jax.ShapeDtypeStruct((_SEQ, _DIM, _BATCH), jnp.float32),
        mesh=mesh,
        scratch_types=[
            pltpu.VMEM((_SEQ, _BPW), jnp.int32),
            pltpu.VMEM((_DIM, _PSTRIDE), jnp.float32),
            pltpu.VMEM((_BPW, _DIM), jnp.float32),
            pltpu.VMEM((_BPW, _DIM), jnp.float32),
            pltpu.VMEM((_DIM, _OSTRIDE), jnp.float32),
            pltpu.VMEM((_DIM, _OSTRIDE), jnp.float32),
            pltpu.SemaphoreType.DMA,
            pltpu.SemaphoreType.DMA,
            pltpu.SemaphoreType.DMA,
            pltpu.SemaphoreType.DMA,
        ],
        compiler_params=pltpu.CompilerParams(
            use_tc_tiling_on_sc=False, needs_layout_passes=False),
    )(idx_t, word_lin, pos_tt)

    return out_t.transpose(2, 0, 1)


def kernel(inputs, word_table, pos_table):
    return _embed(inputs, word_table, pos_table)


# speedup vs baseline: 1.1043x; 1.1043x over previous
import jax
import jax.numpy as jnp
from jax import lax
from jax.experimental import pallas as pl
from jax.experimental.pallas import tpu as pltpu
from jax.experimental.pallas import tpu_sc as plsc

_VOCAB = 1000000
_SEQ = 200
_DIM = 32
_BATCH = 4096

_NC = 2
_NS = 16
_NW = _NC * _NS
_BPW = _BATCH // _NW
_OSTRIDE = _BPW + 1
_PSTRIDE = _SEQ + 1


_TBLK = 512
_NBLK = 489
_PACK = _TBLK * _NBLK


def _pack_body(x0, x1, x2, x3, o_ref):
    for a, x in enumerate((x0, x1, x2, x3)):
        o_ref[:, 32 * a:32 * (a + 1)] = x[...].T


def _gather_body(idx_t, word_hbm, pos_tt, out_t,
                 idx_v, pos_v, g0, g1, o0, o1, gsem0, gsem1, osem0, osem1):
    wid = lax.axis_index("s") * _NC + lax.axis_index("c")
    b0 = wid * _BPW
    grows = (g0, g1)
    oblk = (o0, o1)
    gsem = (gsem0, gsem1)
    osem = (osem0, osem1)

    pltpu.sync_copy(idx_t.at[:, pl.ds(b0, _BPW)], idx_v)
    pltpu.sync_copy(pos_tt, pos_v.at[:, pl.ds(0, _SEQ)])

    def translate_row(sr, carry):
        for u in range(_BPW // 16):
            vec = idx_v[sr, pl.ds(16 * u, 16)]
            a = (jnp.where(vec >= _PACK, 1, 0)
                 + jnp.where(vec >= 2 * _PACK, 1, 0)
                 + jnp.where(vec >= 3 * _PACK, 1, 0))
            idx_v[sr, pl.ds(16 * u, 16)] = 4 * vec - a * (4 * _PACK - 1)
        return carry

    lax.fori_loop(0, _SEQ, translate_row, 0, unroll=2)

    def g_copy(s, b):
        return pltpu.make_async_copy(
            word_hbm.at[idx_v.at[s]],
            grows[b],
            gsem[b],
        )

    def o_copy(s, b):
        return pltpu.make_async_copy(
            oblk[b].at[:, pl.ds(0, _BPW)],
            out_t.at[s, :, pl.ds(b0, _BPW)],
            osem[b],
        )

    lanes = lax.iota(jnp.int32, 16)

    def compute(s, b):
        gb, ob = grows[b], oblk[b]
        ss = jnp.full((16,), s, jnp.int32)
        pcol = []
        for h in range(2):
            cc = lanes + (16 * h)
            pcol.append(plsc.load_gather(pos_v, [cc, ss]))
        def rbody(r):
            rr = jnp.full((16,), r, jnp.int32)
            for h in range(2):
                cc = lanes + (16 * h)
                vec = gb[r, pl.ds(16 * h, 16)]
                plsc.store_scatter(ob, [cc, rr], vec + pcol[h])

        plsc.parallel_loop(0, _BPW, 1, unroll=8)(rbody)

    g_copy(0, 0).start()

    def s_iter(t, carry):
        for b in range(2):
            s = t * 2 + b
            nb = 1 - b

            @pl.when(s + 1 < _SEQ)
            def _prefetch():
                @pl.when(s >= 1)
                def _drain():
                    o_copy(s - 1, nb).wait()

                g_copy(s + 1, nb).start()

            g_copy(s, b).wait()
            compute(s, b)
            o_copy(s, b).start()
        return carry

    lax.fori_loop(0, _SEQ // 2, s_iter, 0)

    o_copy(_SEQ - 2, 0).wait()
    o_copy(_SEQ - 1, 1).wait()


@jax.jit
def _embed(inputs, word_table, pos_table):
    mesh = plsc.VectorSubcoreMesh(core_axis_name="c", subcore_axis_name="s")
    idx_t = inputs.T
    pos_tt = pos_table.T

    wt = word_table.T
    word128 = pl.pallas_call(
        _pack_body,
        grid=(_NBLK,),
        in_specs=[
            pl.BlockSpec(
                (_DIM, _TBLK),
                lambda i, a=a: (0, jnp.minimum(i + a * _NBLK,
                                               _VOCAB // _TBLK)))
            for a in range(4)
        ],
        out_specs=pl.BlockSpec((_TBLK, 128), lambda i: (i, 0)),
        out_shape=jax.ShapeDtypeStruct((_PACK, 128), jnp.float32),
    )(wt, wt, wt, wt)
    word_lin = word128.reshape(4 * _PACK, _DIM)

    out_t = pl.kernel(
        _gather_body,
        out_type=jax.ShapeDtypeStruct((_SEQ, _DIM, _BATCH), jnp.float32),
        mesh=mesh,
        scratch_types=[
            pltpu.VMEM((_SEQ, _BPW), jnp.int32),
            pltpu.VMEM((_DIM, _PSTRIDE), jnp.float32),
            pltpu.VMEM((_BPW, _DIM), jnp.float32),
            pltpu.VMEM((_BPW, _DIM), jnp.float32),
            pltpu.VMEM((_DIM, _OSTRIDE), jnp.float32),
            pltpu.VMEM((_DIM, _OSTRIDE), jnp.float32),
            pltpu.SemaphoreType.DMA,
            pltpu.SemaphoreType.DMA,
            pltpu.SemaphoreType.DMA,
            pltpu.SemaphoreType.DMA,
        ],
        compiler_params=pltpu.CompilerParams(
            use_tc_tiling_on_sc=False, needs_layout_passes=False),
    )(idx_t, word_lin, pos_tt)

    return out_t.transpose(2, 0, 1)


def kernel(inputs, word_table, pos_table):
    return _embed(inputs, word_table, pos_table)


# speedup vs baseline: 1.4278x; 1.2930x over previous
import jax
import jax.numpy as jnp
from jax import lax
from jax.experimental import pallas as pl
from jax.experimental.pallas import tpu as pltpu
from jax.experimental.pallas import tpu_sc as plsc

_VOCAB = 1000000
_SEQ = 200
_DIM = 32
_BATCH = 4096

_NC = 2
_NS = 16
_NW = _NC * _NS
_BPW = _BATCH // _NW
_OSTRIDE = _BPW + 1
_PSTRIDE = _SEQ + 1


_TBLK = 2048
_NBLK = 123
_PACK = _TBLK * _NBLK


def _pack_body(x0, x1, x2, x3, o_ref):
    for a, x in enumerate((x0, x1, x2, x3)):
        o_ref[:, 32 * a:32 * (a + 1)] = x[...].T


def _gather_body(idx_t, word_hbm, pos_tt, out_t,
                 idx_v, pos_v, g0, g1, o0, o1, gsem0, gsem1, osem0, osem1):
    wid = lax.axis_index("s") * _NC + lax.axis_index("c")
    b0 = wid * _BPW
    grows = (g0, g1)
    oblk = (o0, o1)
    gsem = (gsem0, gsem1)
    osem = (osem0, osem1)

    pltpu.sync_copy(idx_t.at[:, pl.ds(b0, _BPW)], idx_v)
    pltpu.sync_copy(pos_tt, pos_v.at[:, pl.ds(0, _SEQ)])

    def translate_row(sr, carry):
        for u in range(_BPW // 16):
            vec = idx_v[sr, pl.ds(16 * u, 16)]
            a = (jnp.where(vec >= _PACK, 1, 0)
                 + jnp.where(vec >= 2 * _PACK, 1, 0)
                 + jnp.where(vec >= 3 * _PACK, 1, 0))
            idx_v[sr, pl.ds(16 * u, 16)] = 4 * vec - a * (4 * _PACK - 1)
        return carry

    lax.fori_loop(0, _SEQ, translate_row, 0, unroll=2)

    def g_copy(s, b):
        return pltpu.make_async_copy(
            word_hbm.at[idx_v.at[s]],
            grows[b],
            gsem[b],
        )

    def o_copy(s, b):
        return pltpu.make_async_copy(
            oblk[b].at[:, pl.ds(0, _BPW)],
            out_t.at[s, :, pl.ds(b0, _BPW)],
            osem[b],
        )

    lanes = lax.iota(jnp.int32, 16)

    def compute(s, b):
        gb, ob = grows[b], oblk[b]
        ss = jnp.full((16,), s, jnp.int32)
        pcol = []
        for h in range(2):
            cc = lanes + (16 * h)
            pcol.append(plsc.load_gather(pos_v, [cc, ss]))
        def rbody(r):
            rr = jnp.full((16,), r, jnp.int32)
            for h in range(2):
                cc = lanes + (16 * h)
                vec = gb[r, pl.ds(16 * h, 16)]
                plsc.store_scatter(ob, [cc, rr], vec + pcol[h])

        plsc.parallel_loop(0, _BPW, 1, unroll=8)(rbody)

    g_copy(0, 0).start()

    def s_iter(t, carry):
        for b in range(2):
            s = t * 2 + b
            nb = 1 - b

            @pl.when(s + 1 < _SEQ)
            def _prefetch():
                @pl.when(s >= 1)
                def _drain():
                    o_copy(s - 1, nb).wait()

                g_copy(s + 1, nb).start()

            g_copy(s, b).wait()
            compute(s, b)
            o_copy(s, b).start()
        return carry

    lax.fori_loop(0, _SEQ // 2, s_iter, 0)

    o_copy(_SEQ - 2, 0).wait()
    o_copy(_SEQ - 1, 1).wait()


@jax.jit
def _embed(inputs, word_table, pos_table):
    mesh = plsc.VectorSubcoreMesh(core_axis_name="c", subcore_axis_name="s")
    idx_t = inputs.T
    pos_tt = pos_table.T

    wt = word_table.T
    word128 = pl.pallas_call(
        _pack_body,
        grid=(_NBLK,),
        in_specs=[
            pl.BlockSpec(
                (_DIM, _TBLK),
                lambda i, a=a: (0, jnp.minimum(i + a * _NBLK,
                                               _VOCAB // _TBLK)))
            for a in range(4)
        ],
        out_specs=pl.BlockSpec((_TBLK, 128), lambda i: (i, 0)),
        out_shape=jax.ShapeDtypeStruct((_PACK, 128), jnp.float32),
    )(wt, wt, wt, wt)
    word_lin = word128.reshape(4 * _PACK, _DIM)

    out_t = pl.kernel(
        _gather_body,
        out_type=jax.ShapeDtypeStruct((_SEQ, _DIM, _BATCH), jnp.float32),
        mesh=mesh,
        scratch_types=[
            pltpu.VMEM((_SEQ, _BPW), jnp.int32),
            pltpu.VMEM((_DIM, _PSTRIDE), jnp.float32),
            pltpu.VMEM((_BPW, _DIM), jnp.float32),
            pltpu.VMEM((_BPW, _DIM), jnp.float32),
            pltpu.VMEM((_DIM, _OSTRIDE), jnp.float32),
            pltpu.VMEM((_DIM, _OSTRIDE), jnp.float32),
            pltpu.SemaphoreType.DMA,
            pltpu.SemaphoreType.DMA,
            pltpu.SemaphoreType.DMA,
            pltpu.SemaphoreType.DMA,
        ],
        compiler_params=pltpu.CompilerParams(
            use_tc_tiling_on_sc=False, needs_layout_passes=False),
    )(idx_t, word_lin, pos_tt)

    return out_t.transpose(2, 0, 1)


def kernel(inputs, word_table, pos_table):
    return _embed(inputs, word_table, pos_table)


# speedup vs baseline: 1.4519x; 1.0169x over previous
import jax
import jax.numpy as jnp
from jax import lax
from jax.experimental import pallas as pl
from jax.experimental.pallas import tpu as pltpu
from jax.experimental.pallas import tpu_sc as plsc

_VOCAB = 1000000
_SEQ = 200
_DIM = 32
_BATCH = 4096

_NC = 2
_NS = 16
_NW = _NC * _NS
_BPW = _BATCH // _NW
_OSTRIDE = _BPW + 1
_PSTRIDE = _SEQ + 1


_TBLK = 4096
_NBLK = 62
_PACK = _TBLK * _NBLK


def _pack_body(x0, x1, x2, x3, o_ref):
    for a, x in enumerate((x0, x1, x2, x3)):
        o_ref[:, 32 * a:32 * (a + 1)] = x[...].T


def _gather_body(idx_t, word_hbm, pos_tt, out_t,
                 idx_v, pos_v, g0, g1, o0, o1, gsem0, gsem1, osem0, osem1):
    wid = lax.axis_index("s") * _NC + lax.axis_index("c")
    b0 = wid * _BPW
    grows = (g0, g1)
    oblk = (o0, o1)
    gsem = (gsem0, gsem1)
    osem = (osem0, osem1)

    pltpu.sync_copy(idx_t.at[:, pl.ds(b0, _BPW)], idx_v)
    pltpu.sync_copy(pos_tt, pos_v.at[:, pl.ds(0, _SEQ)])

    def translate_row(sr, carry):
        for u in range(_BPW // 16):
            vec = idx_v[sr, pl.ds(16 * u, 16)]
            a = (jnp.where(vec >= _PACK, 1, 0)
                 + jnp.where(vec >= 2 * _PACK, 1, 0)
                 + jnp.where(vec >= 3 * _PACK, 1, 0))
            idx_v[sr, pl.ds(16 * u, 16)] = 4 * vec - a * (4 * _PACK - 1)
        return carry

    lax.fori_loop(0, _SEQ, translate_row, 0, unroll=2)

    def g_copy(s, b):
        return pltpu.make_async_copy(
            word_hbm.at[idx_v.at[s]],
            grows[b],
            gsem[b],
        )

    def o_copy(s, b):
        return pltpu.make_async_copy(
            oblk[b].at[:, pl.ds(0, _BPW)],
            out_t.at[s, :, pl.ds(b0, _BPW)],
            osem[b],
        )

    lanes = lax.iota(jnp.int32, 16)

    def compute(s, b):
        gb, ob = grows[b], oblk[b]
        ss = jnp.full((16,), s, jnp.int32)
        pcol = []
        for h in range(2):
            cc = lanes + (16 * h)
            pcol.append(plsc.load_gather(pos_v, [cc, ss]))
        def rbody(r):
            rr = jnp.full((16,), r, jnp.int32)
            for h in range(2):
                cc = lanes + (16 * h)
                vec = gb[r, pl.ds(16 * h, 16)]
                plsc.store_scatter(ob, [cc, rr], vec + pcol[h])

        plsc.parallel_loop(0, _BPW, 1, unroll=8)(rbody)

    g_copy(0, 0).start()

    def s_iter(t, carry):
        for b in range(2):
            s = t * 2 + b
            nb = 1 - b

            @pl.when(s + 1 < _SEQ)
            def _prefetch():
                @pl.when(s >= 1)
                def _drain():
                    o_copy(s - 1, nb).wait()

                g_copy(s + 1, nb).start()

            g_copy(s, b).wait()
            compute(s, b)
            o_copy(s, b).start()
        return carry

    lax.fori_loop(0, _SEQ // 2, s_iter, 0)

    o_copy(_SEQ - 2, 0).wait()
    o_copy(_SEQ - 1, 1).wait()


@jax.jit
def _embed(inputs, word_table, pos_table):
    mesh = plsc.VectorSubcoreMesh(core_axis_name="c", subcore_axis_name="s")
    idx_t = inputs.T
    pos_tt = pos_table.T

    wt = word_table.T
    word128 = pl.pallas_call(
        _pack_body,
        grid=(_NBLK,),
        in_specs=[
            pl.BlockSpec(
                (_DIM, _TBLK),
                lambda i, a=a: (0, jnp.minimum(i + a * _NBLK,
                                               _VOCAB // _TBLK)))
            for a in range(4)
        ],
        out_specs=pl.BlockSpec((_TBLK, 128), lambda i: (i, 0)),
        out_shape=jax.ShapeDtypeStruct((_PACK, 128), jnp.float32),
    )(wt, wt, wt, wt)
    word_lin = word128.reshape(4 * _PACK, _DIM)

    out_t = pl.kernel(
        _gather_body,
        out_type=jax.ShapeDtypeStruct((_SEQ, _DIM, _BATCH), jnp.float32),
        mesh=mesh,
        scratch_types=[
            pltpu.VMEM((_SEQ, _BPW), jnp.int32),
            pltpu.VMEM((_DIM, _PSTRIDE), jnp.float32),
            pltpu.VMEM((_BPW, _DIM), jnp.float32),
            pltpu.VMEM((_BPW, _DIM), jnp.float32),
            pltpu.VMEM((_DIM, _OSTRIDE), jnp.float32),
            pltpu.VMEM((_DIM, _OSTRIDE), jnp.float32),
            pltpu.SemaphoreType.DMA,
            pltpu.SemaphoreType.DMA,
            pltpu.SemaphoreType.DMA,
            pltpu.SemaphoreType.DMA,
        ],
        compiler_params=pltpu.CompilerParams(
            use_tc_tiling_on_sc=False, needs_layout_passes=False),
    )(idx_t, word_lin, pos_tt)

    return out_t.transpose(2, 0, 1)


def kernel(inputs, word_table, pos_table):
    return _embed(inputs, word_table, pos_table)


# speedup vs baseline: 1.8245x; 1.2566x over previous
import jax
import jax.numpy as jnp
from jax import lax
from jax.experimental import pallas as pl
from jax.experimental.pallas import tpu as pltpu
from jax.experimental.pallas import tpu_sc as plsc

_VOCAB = 1000000
_SEQ = 200
_DIM = 32
_BATCH = 4096

_NC = 2
_NS = 16
_NW = _NC * _NS
_BPW = _BATCH // _NW
_OSTRIDE = _BPW + 1
_PSTRIDE = _SEQ + 1


_TBLK = 4096
_NBLK = 62
_PACK = _TBLK * _NBLK


def _pack_body(x0, x1, x2, x3, o_ref):
    for a, x in enumerate((x0, x1, x2, x3)):
        o_ref[:, 32 * a:32 * (a + 1)] = x[...].T


def _gather_body(idx_t, word_hbm, pos_tt, out_t,
                 idx_v, pos_v, g0, g1, o0, o1, gsem0, gsem1, osem0, osem1):
    wid = lax.axis_index("s") * _NC + lax.axis_index("c")
    b0 = wid * _BPW
    grows = (g0, g1)
    oblk = (o0, o1)
    gsem = (gsem0, gsem1)
    osem = (osem0, osem1)

    pltpu.sync_copy(idx_t.at[:, pl.ds(b0, _BPW)], idx_v)
    pltpu.sync_copy(pos_tt, pos_v.at[:, pl.ds(0, _SEQ)])

    def translate_row(sr, carry):
        for u in range(_BPW // 16):
            vec = idx_v[sr, pl.ds(16 * u, 16)]
            a = (jnp.where(vec >= _PACK, 1, 0)
                 + jnp.where(vec >= 2 * _PACK, 1, 0)
                 + jnp.where(vec >= 3 * _PACK, 1, 0))
            idx_v[sr, pl.ds(16 * u, 16)] = 4 * vec - a * (4 * _PACK - 1)
        return carry

    lax.fori_loop(0, _SEQ, translate_row, 0, unroll=2)

    def g_copy(s, b):
        return pltpu.make_async_copy(
            word_hbm.at[idx_v.at[s]],
            grows[b],
            gsem[b],
        )

    def _o_copies(s, b):
        return [
            pltpu.make_async_copy(
                oblk[b].at[pl.ds(8 * cb, 8), pl.ds(0, _BPW)],
                out_t.at[s, cb, wid],
                osem[b],
            )
            for cb in range(4)
        ]

    class _OCopy:
        def __init__(self, s, b):
            self._c = _o_copies(s, b)

        def start(self):
            for c in self._c:
                c.start()

        def wait(self):
            for c in self._c:
                c.wait()

    def o_copy(s, b):
        return _OCopy(s, b)

    lanes = lax.iota(jnp.int32, 16)

    def compute(s, b):
        gb, ob = grows[b], oblk[b]
        ss = jnp.full((16,), s, jnp.int32)
        pcol = []
        for h in range(2):
            cc = lanes + (16 * h)
            pcol.append(plsc.load_gather(pos_v, [cc, ss]))
        def rbody(r):
            rr = jnp.full((16,), r, jnp.int32)
            for h in range(2):
                cc = lanes + (16 * h)
                vec = gb[r, pl.ds(16 * h, 16)]
                plsc.store_scatter(ob, [cc, rr], vec + pcol[h])

        plsc.parallel_loop(0, _BPW, 1, unroll=8)(rbody)

    g_copy(0, 0).start()

    def s_iter(t, carry):
        for b in range(2):
            s = t * 2 + b
            nb = 1 - b

            @pl.when(s + 1 < _SEQ)
            def _prefetch():
                @pl.when(s >= 1)
                def _drain():
                    o_copy(s - 1, nb).wait()

                g_copy(s + 1, nb).start()

            g_copy(s, b).wait()
            compute(s, b)
            o_copy(s, b).start()
        return carry

    lax.fori_loop(0, _SEQ // 2, s_iter, 0)

    o_copy(_SEQ - 2, 0).wait()
    o_copy(_SEQ - 1, 1).wait()


@jax.jit
def _embed(inputs, word_table, pos_table):
    mesh = plsc.VectorSubcoreMesh(core_axis_name="c", subcore_axis_name="s")
    idx_t = inputs.T
    pos_tt = pos_table.T

    wt = word_table.T
    word128 = pl.pallas_call(
        _pack_body,
        grid=(_NBLK,),
        in_specs=[
            pl.BlockSpec(
                (_DIM, _TBLK),
                lambda i, a=a: (0, jnp.minimum(i + a * _NBLK,
                                               _VOCAB // _TBLK)))
            for a in range(4)
        ],
        out_specs=pl.BlockSpec((_TBLK, 128), lambda i: (i, 0)),
        out_shape=jax.ShapeDtypeStruct((_PACK, 128), jnp.float32),
    )(wt, wt, wt, wt)
    word_lin = word128.reshape(4 * _PACK, _DIM)

    out_t = pl.kernel(
        _gather_body,
        out_type=jax.ShapeDtypeStruct((_SEQ, 4, _NW, 8, 128), jnp.float32),
        mesh=mesh,
        scratch_types=[
            pltpu.VMEM((_SEQ, _BPW), jnp.int32),
            pltpu.VMEM((_DIM, _PSTRIDE), jnp.float32),
            pltpu.VMEM((_BPW, _DIM), jnp.float32),
            pltpu.VMEM((_BPW, _DIM), jnp.float32),
            pltpu.VMEM((_DIM, _OSTRIDE), jnp.float32),
            pltpu.VMEM((_DIM, _OSTRIDE), jnp.float32),
            pltpu.SemaphoreType.DMA,
            pltpu.SemaphoreType.DMA,
            pltpu.SemaphoreType.DMA,
            pltpu.SemaphoreType.DMA,
        ],
        compiler_params=pltpu.CompilerParams(
            use_tc_tiling_on_sc=False, needs_layout_passes=False),
    )(idx_t, word_lin, pos_tt)

    return out_t.transpose(2, 4, 0, 1, 3).reshape(_BATCH, _SEQ, _DIM)


def kernel(inputs, word_table, pos_table):
    return _embed(inputs, word_table, pos_table)
